# baseline (speedup 1.0000x reference)
import jax
import jax.numpy as jnp
from jax.experimental import pallas as pl

N_EXPERTS = 64
K = 8
HID = 2048
BLK = 2048
SUB = 2
SBLK = BLK // SUB


def _router_kernel(x_ref, w_ref, logits_ref, scores_ref, idx_ref):
    j = pl.program_id(1)
    x = x_ref[pl.ds(j * SBLK, SBLK), :]
    w = w_ref[...]
    lt = jax.lax.dot_general(
        w, x, (((1,), (1,)), ((), ())), preferred_element_type=jnp.float32
    )
    logits_ref[...] = lt.T

    iota = jax.lax.broadcasted_iota(jnp.int32, lt.shape, 0).astype(jnp.float32)
    vals = []
    idxs = []
    cur = lt
    for _ in range(K):
        mv = jnp.max(cur, axis=0, keepdims=True)
        mi = jnp.min(jnp.where(cur == mv, iota, 64.0), axis=0, keepdims=True)
        vals.append(mv)
        idxs.append(mi)
        cur = jnp.where(iota == mi, -jnp.inf, cur)
    v = jnp.concatenate(vals, axis=0)
    e = jnp.exp(v - v[0:1, :])
    s = e / jnp.sum(e, axis=0, keepdims=True)
    scores_ref[...] = s.T
    idx_ref[...] = jnp.concatenate(idxs, axis=0).T.astype(jnp.int32)


def kernel(hidden_states, weight):
    n = hidden_states.shape[0]
    outs = pl.pallas_call(
        _router_kernel,
        grid=(n // BLK, SUB),
        in_specs=[
            pl.BlockSpec((BLK, HID), lambda i, j: (i, 0)),
            pl.BlockSpec((N_EXPERTS, HID), lambda i, j: (0, 0)),
        ],
        out_specs=[
            pl.BlockSpec((SBLK, N_EXPERTS), lambda i, j: (SUB * i + j, 0)),
            pl.BlockSpec((SBLK, K), lambda i, j: (SUB * i + j, 0)),
            pl.BlockSpec((SBLK, K), lambda i, j: (SUB * i + j, 0)),
        ],
        out_shape=[
            jax.ShapeDtypeStruct((n, N_EXPERTS), jnp.float32),
            jax.ShapeDtypeStruct((n, K), jnp.float32),
            jax.ShapeDtypeStruct((n, K), jnp.int32),
        ],
    )(hidden_states, weight)
    return (outs[0], outs[1], outs[2])


# speedup vs baseline: 1.3107x; 1.3107x over previous
import jax
import jax.numpy as jnp
from jax.experimental import pallas as pl

N_EXPERTS = 64
K = 8
HID = 2048
BLK = 2048


def _router_kernel(x_ref, w_ref, logits_ref, scores_ref, idx_ref):
    x = x_ref[...]
    w = w_ref[...]
    lt = jax.lax.dot_general(
        w, x, (((1,), (1,)), ((), ())), preferred_element_type=jnp.float32
    )
    logits_ref[...] = lt.T

    iota = jax.lax.broadcasted_iota(jnp.int32, lt.shape, 0).astype(jnp.float32)
    vals = []
    idxs = []
    cur = lt
    for _ in range(K):
        mv = jnp.max(cur, axis=0, keepdims=True)
        mi = jnp.min(jnp.where(cur == mv, iota, 64.0), axis=0, keepdims=True)
        vals.append(mv)
        idxs.append(mi)
        cur = jnp.where(iota == mi, -jnp.inf, cur)
    v = jnp.concatenate(vals, axis=0)
    e = jnp.exp(v - v[0:1, :])
    s = e / jnp.sum(e, axis=0, keepdims=True)
    scores_ref[...] = s.T
    idx_ref[...] = jnp.concatenate(idxs, axis=0).T.astype(jnp.int32)


def kernel(hidden_states, weight):
    n = hidden_states.shape[0]
    outs = pl.pallas_call(
        _router_kernel,
        grid=(n // BLK,),
        in_specs=[
            pl.BlockSpec((BLK, HID), lambda i: (i, 0)),
            pl.BlockSpec((N_EXPERTS, HID), lambda i: (0, 0)),
        ],
        out_specs=[
            pl.BlockSpec((BLK, N_EXPERTS), lambda i: (i, 0)),
            pl.BlockSpec((BLK, K), lambda i: (i, 0)),
            pl.BlockSpec((BLK, K), lambda i: (i, 0)),
        ],
        out_shape=[
            jax.ShapeDtypeStruct((n, N_EXPERTS), jnp.float32),
            jax.ShapeDtypeStruct((n, K), jnp.float32),
            jax.ShapeDtypeStruct((n, K), jnp.int32),
        ],
    )(hidden_states, weight)
    return (outs[0], outs[1], outs[2])
